# baseline (speedup 1.0000x reference)
import functools

import jax
import jax.numpy as jnp
from jax import lax
from jax.experimental import pallas as pl
from jax.experimental.pallas import tpu as pltpu
from jax.experimental.pallas import tpu_sc as plsc

N_ROWS = 16384
N_CODES = 8192
DIM = 256
BETA_ = 1.0

BZ = 512
BW = 8192
NZ = N_ROWS // BZ
NWB = N_CODES // BW
LOSS_SCALE = (1.0 + BETA_) / (N_ROWS * DIM)


_LANES = 128
_NCOL = BW // _LANES


def _dist_argmin_body(z_ref, w_ref, idx_ref, loss_ref, wn_s, w2_s):
    i = pl.program_id(0)

    @pl.when(i == 0)
    def _():
        w = w_ref[...]
        w2_s[...] = w + w
        wn_s[...] = jnp.sum(w * w, axis=1).reshape(1, N_CODES)

    z0 = z_ref[...]
    zn = jnp.sum(z0 * z0, axis=1, keepdims=True)
    mm2 = lax.dot_general(z0, w2_s[...],
                          dimension_numbers=(((1,), (1,)), ((), ())),
                          preferred_element_type=jnp.float32)
    dd = (zn + wn_s[...]) - mm2

    lane = lax.broadcasted_iota(jnp.int32, (BZ, _LANES), 1)
    cur = [(dd[:, k * _LANES:(k + 1) * _LANES], lane + k * _LANES)
           for k in range(_NCOL)]
    while len(cur) > 1:
        nxt = []
        for k in range(0, len(cur), 2):
            (va, ca), (vb, cb) = cur[k], cur[k + 1]
            upd = vb < va
            nxt.append((jnp.where(upd, vb, va), jnp.where(upd, cb, ca)))
        cur = nxt
    fv, fc = cur[0]

    gmin = jnp.min(fv, axis=1)
    idx_ref[0, 0, :] = jnp.min(
        jnp.where(fv == gmin[:, None], fc, jnp.int32(2**31 - 1)), axis=1)
    part = jnp.sum(gmin).reshape(1, 1)
    prev = jnp.where(i == 0, jnp.zeros((1, 1), jnp.float32), loss_ref[...])
    tot = prev + part
    loss_ref[...] = jnp.where(i == NZ - 1, tot * LOSS_SCALE, tot)


def _dist_argmin(z, W):
    return pl.pallas_call(
        _dist_argmin_body,
        grid=(NZ,),
        in_specs=[
            pl.BlockSpec((BZ, DIM), lambda i: (i, 0)),
            pl.BlockSpec((N_CODES, DIM), lambda i: (0, 0)),
        ],
        out_specs=[
            pl.BlockSpec((1, 1, BZ), lambda i: (i, 0, 0)),
            pl.BlockSpec((1, 1), lambda i: (0, 0)),
        ],
        out_shape=[
            jax.ShapeDtypeStruct((NZ, 1, BZ), jnp.int32),
            jax.ShapeDtypeStruct((1, 1), jnp.float32),
        ],
        scratch_shapes=[
            pltpu.VMEM((1, N_CODES), jnp.float32),
            pltpu.VMEM((N_CODES, DIM), jnp.float32),
        ],
        compiler_params=pltpu.CompilerParams(
            dimension_semantics=("arbitrary",)),
    )(z, W)


_NC = 2
_NS = 16
_NWK = _NC * _NS
_BPW = N_ROWS // _NWK
_CH = 128
_NCH = _BPW // _CH


def _sc_gather(W, idx):
    mesh = plsc.VectorSubcoreMesh(core_axis_name="c", subcore_axis_name="s")

    @functools.partial(
        pl.kernel, mesh=mesh,
        out_type=jax.ShapeDtypeStruct((N_ROWS, DIM), jnp.float32),
        scratch_types=[
            pltpu.VMEM((_CH,), jnp.int32),
            pltpu.VMEM((_CH, DIM), jnp.float32),
            pltpu.SemaphoreType.DMA,
        ],
    )
    def k(table_hbm, idx_hbm, out_hbm, idx_v, rows_v, sem):
        wid = lax.axis_index("s") * _NC + lax.axis_index("c")
        base = wid * _BPW
        for c in range(_NCH):
            off = base + c * _CH
            pltpu.sync_copy(idx_hbm.at[pl.ds(off, _CH)], idx_v)
            pltpu.async_copy(table_hbm.at[idx_v], rows_v, sem).wait()
            pltpu.sync_copy(rows_v, out_hbm.at[pl.ds(off, _CH)])

    return k(W, idx)


def kernel(z, W):
    idx3, loss2 = _dist_argmin(z, W)
    idx = idx3.reshape(N_ROWS)
    z_q = _sc_gather(W, idx)
    loss = loss2[0, 0]
    return (loss, z_q, idx)


# speedup vs baseline: 1.0021x; 1.0021x over previous
import functools

import jax
import jax.numpy as jnp
from jax import lax
from jax.experimental import pallas as pl
from jax.experimental.pallas import tpu as pltpu
from jax.experimental.pallas import tpu_sc as plsc

N_ROWS = 16384
N_CODES = 8192
DIM = 256
BETA_ = 1.0

BZ = 512
BW = 8192
NZ = N_ROWS // BZ
NWB = N_CODES // BW
LOSS_SCALE = (1.0 + BETA_) / (N_ROWS * DIM)


_LANES = 128
_NCOL = BW // _LANES


def _dist_argmin_body(z_ref, w_ref, idx_ref, loss_ref, wn_s, w2_s):
    i = pl.program_id(0)

    @pl.when(i == 0)
    def _():
        w = w_ref[...]
        w2_s[...] = w + w
        wn_s[...] = jnp.sum(w * w, axis=1).reshape(1, N_CODES)

    z0 = z_ref[...]
    zn = jnp.sum(z0 * z0, axis=1, keepdims=True)
    zn128 = jnp.broadcast_to(zn, (BZ, _LANES))
    mm2 = lax.dot_general(z0, w2_s[...],
                          dimension_numbers=(((1,), (1,)), ((), ())),
                          preferred_element_type=jnp.float32)

    lane = lax.broadcasted_iota(jnp.int32, (BZ, _LANES), 1)
    cur = [((zn128 + wn_s[:, k * _LANES:(k + 1) * _LANES])
            - mm2[:, k * _LANES:(k + 1) * _LANES], lane + k * _LANES)
           for k in range(_NCOL)]
    while len(cur) > 1:
        nxt = []
        for k in range(0, len(cur), 2):
            (va, ca), (vb, cb) = cur[k], cur[k + 1]
            upd = vb < va
            nxt.append((jnp.where(upd, vb, va), jnp.where(upd, cb, ca)))
        cur = nxt
    fv, fc = cur[0]

    gmin = jnp.min(fv, axis=1)
    idx_ref[0, 0, :] = jnp.min(
        jnp.where(fv == gmin[:, None], fc, jnp.int32(2**31 - 1)), axis=1)
    part = jnp.sum(gmin).reshape(1, 1)
    prev = jnp.where(i == 0, jnp.zeros((1, 1), jnp.float32), loss_ref[...])
    tot = prev + part
    loss_ref[...] = jnp.where(i == NZ - 1, tot * LOSS_SCALE, tot)


def _dist_argmin(z, W):
    return pl.pallas_call(
        _dist_argmin_body,
        grid=(NZ,),
        in_specs=[
            pl.BlockSpec((BZ, DIM), lambda i: (i, 0)),
            pl.BlockSpec((N_CODES, DIM), lambda i: (0, 0)),
        ],
        out_specs=[
            pl.BlockSpec((1, 1, BZ), lambda i: (i, 0, 0)),
            pl.BlockSpec((1, 1), lambda i: (0, 0)),
        ],
        out_shape=[
            jax.ShapeDtypeStruct((NZ, 1, BZ), jnp.int32),
            jax.ShapeDtypeStruct((1, 1), jnp.float32),
        ],
        scratch_shapes=[
            pltpu.VMEM((1, N_CODES), jnp.float32),
            pltpu.VMEM((N_CODES, DIM), jnp.float32),
        ],
        compiler_params=pltpu.CompilerParams(
            dimension_semantics=("arbitrary",)),
    )(z, W)


_NC = 2
_NS = 16
_NWK = _NC * _NS
_BPW = N_ROWS // _NWK
_CH = 128
_NCH = _BPW // _CH


def _sc_gather(W, idx):
    mesh = plsc.VectorSubcoreMesh(core_axis_name="c", subcore_axis_name="s")

    @functools.partial(
        pl.kernel, mesh=mesh,
        out_type=jax.ShapeDtypeStruct((N_ROWS, DIM), jnp.float32),
        scratch_types=[
            pltpu.VMEM((_CH,), jnp.int32),
            pltpu.VMEM((_CH, DIM), jnp.float32),
            pltpu.SemaphoreType.DMA,
        ],
    )
    def k(table_hbm, idx_hbm, out_hbm, idx_v, rows_v, sem):
        wid = lax.axis_index("s") * _NC + lax.axis_index("c")
        base = wid * _BPW
        for c in range(_NCH):
            off = base + c * _CH
            pltpu.sync_copy(idx_hbm.at[pl.ds(off, _CH)], idx_v)
            pltpu.async_copy(table_hbm.at[idx_v], rows_v, sem).wait()
            pltpu.sync_copy(rows_v, out_hbm.at[pl.ds(off, _CH)])

    return k(W, idx)


def kernel(z, W):
    idx3, loss2 = _dist_argmin(z, W)
    idx = idx3.reshape(N_ROWS)
    z_q = _sc_gather(W, idx)
    loss = loss2[0, 0]
    return (loss, z_q, idx)
